# baseline (speedup 1.0000x reference)
import functools

import jax
import jax.numpy as jnp
from jax import lax
from jax.experimental import pallas as pl
from jax.experimental.pallas import tpu as pltpu
from jax.experimental.pallas import tpu_sc as plsc

BLK = 256
NC, NS = 2, 16
NW = NC * NS


def _plan_kernel(nb_total, x_ref, gw_ref,
                 pos1_ref, pos2_ref, w1_ref, w2_ref, bexp_ref):
    T, _ = x_ref.shape
    E = gw_ref.shape[0]
    xb = x_ref[...].astype(jnp.bfloat16)
    gw = gw_ref[...].astype(jnp.bfloat16)
    logits = lax.dot_general(xb, gw, (((1,), (1,)), ((), ())),
                             preferred_element_type=jnp.float32)
    cols = lax.broadcasted_iota(jnp.int32, (T, E), 1)
    m1 = jnp.max(logits, axis=1, keepdims=True)
    i1 = jnp.min(jnp.where(logits == m1, cols, E), axis=1, keepdims=True)
    mask1 = cols == i1
    l2 = jnp.where(mask1, -jnp.inf, logits)
    m2 = jnp.max(l2, axis=1, keepdims=True)
    i2 = jnp.min(jnp.where(l2 == m2, cols, E), axis=1, keepdims=True)
    mask2 = cols == i2
    p1 = 1.0 / (1.0 + jnp.exp(m2 - m1))

    maskf = (mask1 | mask2).astype(jnp.float32)
    incl = maskf
    s = 1
    while s < T:
        shifted = jnp.concatenate(
            [jnp.zeros((s, E), jnp.float32), incl[:T - s]], axis=0)
        incl = incl + shifted
        s *= 2
    rank = incl - maskf
    counts = incl[T - 1:T, :]
    nblk = jnp.ceil(counts / BLK)

    incl8 = nblk
    s = 1
    while s < E:
        incl8 = incl8 + jnp.concatenate(
            [jnp.zeros((1, s), jnp.float32), incl8[:, :E - s]], axis=1)
        s *= 2
    gsb = incl8 - nblk
    posf = gsb * BLK + rank
    pos1 = jnp.sum(jnp.where(mask1, posf, 0.0), axis=1, keepdims=True)
    pos2 = jnp.sum(jnp.where(mask2, posf, 0.0), axis=1, keepdims=True)
    pos1_ref[...] = pos1.astype(jnp.int32)
    pos2_ref[...] = pos2.astype(jnp.int32)
    w1_ref[...] = jnp.broadcast_to(p1, (T, 128))
    w2_ref[...] = jnp.broadcast_to(1.0 - p1, (T, 128))

    geb = (gsb + nblk).astype(jnp.int32)
    nbi = lax.broadcasted_iota(jnp.int32, (nb_total, E), 0)
    be = jnp.sum((nbi >= geb).astype(jnp.int32), axis=1, keepdims=True)
    bexp_ref[...] = jnp.minimum(be, E - 1)


def _shared_kernel(xb_ref, wsg_ref, wsu_ref, wsd_ref, out_ref):
    xb = xb_ref[...]
    gs = lax.dot_general(xb, wsg_ref[...], (((1,), (1,)), ((), ())),
                         preferred_element_type=jnp.float32)
    us = lax.dot_general(xb, wsu_ref[...], (((1,), (1,)), ((), ())),
                         preferred_element_type=jnp.float32)
    hs = (gs * jax.nn.sigmoid(gs) * us).astype(jnp.bfloat16)
    out_ref[...] = lax.dot_general(hs, wsd_ref[...], (((1,), (1,)), ((), ())),
                                   preferred_element_type=jnp.float32)


def _group_kernel(bexp_ref, xs_ref, rw_ref, wg_ref, wu_ref, wd_ref, o_ref):
    del bexp_ref
    xb = xs_ref[...].astype(jnp.bfloat16)
    g = lax.dot_general(xb, wg_ref[0], (((1,), (1,)), ((), ())),
                        preferred_element_type=jnp.float32)
    u = lax.dot_general(xb, wu_ref[0], (((1,), (1,)), ((), ())),
                        preferred_element_type=jnp.float32)
    h = (g * jax.nn.sigmoid(g) * u).astype(jnp.bfloat16)
    o = lax.dot_general(h, wd_ref[0], (((1,), (1,)), ((), ())),
                        preferred_element_type=jnp.float32)
    o_ref[...] = o * rw_ref[:, 0:1]


def _dispatch_body(cpw, x_hbm, pos1_hbm, pos2_hbm, w1_hbm, w2_hbm,
                   xs_hbm, rw_hbm,
                   idx1_v, idx2_v, rows_v, wrow1_v, wrow2_v, sem):
    wid = lax.axis_index("s") * NC + lax.axis_index("c")
    base = wid * cpw
    l1 = pltpu.async_copy(pos1_hbm.at[pl.ds(base, cpw)], idx1_v, sem)
    l2 = pltpu.async_copy(pos2_hbm.at[pl.ds(base, cpw)], idx2_v, sem)
    l3 = pltpu.async_copy(x_hbm.at[pl.ds(base, cpw)], rows_v, sem)
    l4 = pltpu.async_copy(w1_hbm.at[pl.ds(base, cpw)], wrow1_v, sem)
    l5 = pltpu.async_copy(w2_hbm.at[pl.ds(base, cpw)], wrow2_v, sem)
    l1.wait()
    l2.wait()
    l3.wait()
    l4.wait()
    l5.wait()
    c1 = pltpu.async_copy(rows_v, xs_hbm.at[idx1_v], sem)
    c2 = pltpu.async_copy(rows_v, xs_hbm.at[idx2_v], sem)
    c3 = pltpu.async_copy(wrow1_v, rw_hbm.at[idx1_v], sem)
    c4 = pltpu.async_copy(wrow2_v, rw_hbm.at[idx2_v], sem)
    c1.wait()
    c2.wait()
    c3.wait()
    c4.wait()


def _combine_body(cpw, o_hbm, sh_hbm, pos1_hbm, pos2_hbm, y_hbm,
                  idx1_v, idx2_v, r1_v, r2_v, acc_v, sem):
    wid = lax.axis_index("s") * NC + lax.axis_index("c")
    half = cpw // 2
    n_chunk = r1_v.shape[1] // 16
    for r in range(2):
        base = wid * cpw + r * half
        pltpu.sync_copy(pos1_hbm.at[pl.ds(base, half)], idx1_v)
        pltpu.sync_copy(pos2_hbm.at[pl.ds(base, half)], idx2_v)
        c1 = pltpu.async_copy(o_hbm.at[idx1_v], r1_v, sem)
        c2 = pltpu.async_copy(o_hbm.at[idx2_v], r2_v, sem)
        c3 = pltpu.async_copy(sh_hbm.at[pl.ds(base, half)], acc_v, sem)
        c1.wait()
        c2.wait()
        c3.wait()

        @plsc.parallel_loop(0, half * n_chunk, step=1, unroll=8)
        def _chunks(j):
            row = j // n_chunk
            cs = pl.ds((j % n_chunk) * 16, 16)
            acc_v[row, cs] = acc_v[row, cs] + r1_v[row, cs] + r2_v[row, cs]

        pltpu.sync_copy(acc_v, y_hbm.at[pl.ds(base, half)])


def _sc_dispatch(PADTOT, x, pos1f, pos2f, w1rep, w2rep):
    T, D = x.shape
    cpw = T // NW
    mesh = plsc.VectorSubcoreMesh(core_axis_name="c", subcore_axis_name="s",
                                  num_cores=NC, num_subcores=NS)
    fn = pl.kernel(
        functools.partial(_dispatch_body, cpw),
        out_type=(jax.ShapeDtypeStruct((PADTOT, D), jnp.float32),
                  jax.ShapeDtypeStruct((PADTOT, 128), jnp.float32)),
        mesh=mesh,
        scratch_types=[
            pltpu.VMEM((cpw,), jnp.int32),
            pltpu.VMEM((cpw,), jnp.int32),
            pltpu.VMEM((cpw, D), jnp.float32),
            pltpu.VMEM((cpw, 128), jnp.float32),
            pltpu.VMEM((cpw, 128), jnp.float32),
            pltpu.SemaphoreType.DMA,
        ],
    )
    return fn(x, pos1f, pos2f, w1rep, w2rep)


def _sc_combine(o, sh, pos1f, pos2f):
    T, D = sh.shape
    cpw = T // NW
    half = cpw // 2
    mesh = plsc.VectorSubcoreMesh(core_axis_name="c", subcore_axis_name="s",
                                  num_cores=NC, num_subcores=NS)
    fn = pl.kernel(
        functools.partial(_combine_body, cpw),
        out_type=jax.ShapeDtypeStruct((T, D), jnp.float32),
        mesh=mesh,
        scratch_types=[
            pltpu.VMEM((half,), jnp.int32),
            pltpu.VMEM((half,), jnp.int32),
            pltpu.VMEM((half, D), jnp.float32),
            pltpu.VMEM((half, D), jnp.float32),
            pltpu.VMEM((half, D), jnp.float32),
            pltpu.SemaphoreType.DMA,
        ],
    )
    return fn(o, sh, pos1f, pos2f)


def kernel(hidden_states, gate_w, w_gate, w_up, w_down, ws_gate, ws_up, ws_down):
    orig_shape = hidden_states.shape
    x = hidden_states.reshape(-1, orig_shape[-1])
    T, D = x.shape
    E, FF, _ = w_gate.shape
    SFF = ws_gate.shape[0]
    NB = (T * 2) // BLK + E
    PADTOT = NB * BLK
    NTB = 2
    TB = T // NTB

    pos1, pos2, w1rep, w2rep, bexp = pl.pallas_call(
        functools.partial(_plan_kernel, NB),
        out_shape=(
            jax.ShapeDtypeStruct((T, 1), jnp.int32),
            jax.ShapeDtypeStruct((T, 1), jnp.int32),
            jax.ShapeDtypeStruct((T, 128), jnp.float32),
            jax.ShapeDtypeStruct((T, 128), jnp.float32),
            jax.ShapeDtypeStruct((NB, 1), jnp.int32),
        ),
    )(x, gate_w)
    pos1f = pos1.reshape(T)
    pos2f = pos2.reshape(T)
    bexpf = bexp.reshape(NB)

    xb = x.astype(jnp.bfloat16)
    wgb = w_gate.astype(jnp.bfloat16)
    wub = w_up.astype(jnp.bfloat16)
    wdb = w_down.astype(jnp.bfloat16)
    wsgb = ws_gate.astype(jnp.bfloat16)
    wsub = ws_up.astype(jnp.bfloat16)
    wsdb = ws_down.astype(jnp.bfloat16)

    sh = pl.pallas_call(
        _shared_kernel,
        grid=(NTB,),
        in_specs=[
            pl.BlockSpec((TB, D), lambda t: (t, 0)),
            pl.BlockSpec((SFF, D), lambda t: (0, 0)),
            pl.BlockSpec((SFF, D), lambda t: (0, 0)),
            pl.BlockSpec((D, SFF), lambda t: (0, 0)),
        ],
        out_specs=pl.BlockSpec((TB, D), lambda t: (t, 0)),
        out_shape=jax.ShapeDtypeStruct((T, D), jnp.float32),
    )(xb, wsgb, wsub, wsdb)

    xs, rw = _sc_dispatch(PADTOT, x, pos1f, pos2f, w1rep, w2rep)

    o = pl.pallas_call(
        _group_kernel,
        grid_spec=pltpu.PrefetchScalarGridSpec(
            num_scalar_prefetch=1,
            grid=(NB,),
            in_specs=[
                pl.BlockSpec((BLK, D), lambda i, be: (i, 0)),
                pl.BlockSpec((BLK, 128), lambda i, be: (i, 0)),
                pl.BlockSpec((1, FF, D), lambda i, be: (be[i], 0, 0)),
                pl.BlockSpec((1, FF, D), lambda i, be: (be[i], 0, 0)),
                pl.BlockSpec((1, D, FF), lambda i, be: (be[i], 0, 0)),
            ],
            out_specs=pl.BlockSpec((BLK, D), lambda i, be: (i, 0)),
        ),
        out_shape=jax.ShapeDtypeStruct((PADTOT, D), jnp.float32),
    )(bexpf, xs, rw, wgb, wub, wdb)

    y = _sc_combine(o, sh, pos1f, pos2f)
    return y.reshape(orig_shape)


# speedup vs baseline: 1.3972x; 1.3972x over previous
import jax
import jax.numpy as jnp
from jax import lax
from jax.experimental import pallas as pl
from jax.experimental.pallas import tpu as pltpu


def _expert_ffn(xh, wg, wu, wd):
    g = lax.dot_general(xh, wg, (((1,), (1,)), ((), ())),
                        preferred_element_type=jnp.float32)
    u = lax.dot_general(xh, wu, (((1,), (1,)), ((), ())),
                        preferred_element_type=jnp.float32)
    h = (g * jax.nn.sigmoid(g) * u).astype(jnp.bfloat16)
    return lax.dot_general(h, wd, (((1,), (1,)), ((), ())),
                           preferred_element_type=jnp.float32)


def _moe_kernel(xb_ref, gw_ref, wg_ref, wu_ref, wd_ref,
                wsg_ref, wsu_ref, wsd_ref, out_ref, cmb_ref):
    e = pl.program_id(0)
    T, E = cmb_ref.shape
    H = T // 2

    @pl.when(e == 0)
    def _router_and_shared():
        xb = xb_ref[...]
        logits = lax.dot_general(xb, gw_ref[...].astype(jnp.bfloat16),
                                 (((1,), (1,)), ((), ())),
                                 preferred_element_type=jnp.float32)
        cols = lax.broadcasted_iota(jnp.int32, (T, E), 1)
        m1 = jnp.max(logits, axis=1, keepdims=True)
        i1 = jnp.min(jnp.where(logits == m1, cols, E), axis=1, keepdims=True)
        mask1 = cols == i1
        l2 = jnp.where(mask1, -jnp.inf, logits)
        m2 = jnp.max(l2, axis=1, keepdims=True)
        i2 = jnp.min(jnp.where(l2 == m2, cols, E), axis=1, keepdims=True)
        mask2 = cols == i2
        p1 = 1.0 / (1.0 + jnp.exp(m2 - m1))
        cmb_ref[...] = jnp.where(mask1, p1, 0.0) + jnp.where(mask2, 1.0 - p1, 0.0)
        for hh in range(2):
            rows = pl.ds(hh * H, H)
            out_ref[rows, :] = _expert_ffn(xb_ref[rows, :], wsg_ref[...],
                                           wsu_ref[...], wsd_ref[...])

    cmb = cmb_ref[...]
    cols = lax.broadcasted_iota(jnp.int32, (T, E), 1)
    wcol = jnp.sum(jnp.where(cols == e, cmb, 0.0), axis=1, keepdims=True)
    for hh in range(2):
        rows = pl.ds(hh * H, H)
        o = _expert_ffn(xb_ref[rows, :], wg_ref[0], wu_ref[0], wd_ref[0])
        out_ref[rows, :] = out_ref[rows, :] + o * wcol[hh * H:(hh + 1) * H, :]


def kernel(hidden_states, gate_w, w_gate, w_up, w_down, ws_gate, ws_up, ws_down):
    orig_shape = hidden_states.shape
    x = hidden_states.reshape(-1, orig_shape[-1])
    T, D = x.shape
    E, FF, _ = w_gate.shape
    SFF = ws_gate.shape[0]

    xb = x.astype(jnp.bfloat16)
    wg = w_gate.astype(jnp.bfloat16)
    wu = w_up.astype(jnp.bfloat16)
    wd = w_down.astype(jnp.bfloat16)
    wsg = ws_gate.astype(jnp.bfloat16)
    wsu = ws_up.astype(jnp.bfloat16)
    wsd = ws_down.astype(jnp.bfloat16)

    y = pl.pallas_call(
        _moe_kernel,
        grid=(E,),
        in_specs=[
            pl.BlockSpec((T, D), lambda e: (0, 0)),
            pl.BlockSpec((E, D), lambda e: (0, 0)),
            pl.BlockSpec((1, FF, D), lambda e: (e, 0, 0)),
            pl.BlockSpec((1, FF, D), lambda e: (e, 0, 0)),
            pl.BlockSpec((1, D, FF), lambda e: (e, 0, 0)),
            pl.BlockSpec((SFF, D), lambda e: (0, 0)),
            pl.BlockSpec((SFF, D), lambda e: (0, 0)),
            pl.BlockSpec((D, SFF), lambda e: (0, 0)),
        ],
        out_specs=pl.BlockSpec((T, D), lambda e: (0, 0)),
        out_shape=jax.ShapeDtypeStruct((T, D), jnp.float32),
        scratch_shapes=[pltpu.VMEM((T, E), jnp.float32)],
    )(xb, gate_w, wg, wu, wd, wsg, wsu, wsd)

    return y.reshape(orig_shape)
